# baseline (speedup 1.0000x reference)
import functools

import jax
import jax.numpy as jnp
from jax import lax
from jax.experimental import pallas as pl
from jax.experimental.pallas import tpu as pltpu
from jax.experimental.pallas import tpu_sc as plsc


def _partner(x, d):
    n = x.shape[0]
    tail = x.shape[1:]
    y = x.reshape((n // (2 * d), 2, d) + tail)
    y = jnp.concatenate([y[:, 1:2], y[:, 0:1]], axis=1)
    return y.reshape((n,) + tail)


def _stage_mask(iota, k, d):
    bk = (iota >> int(k).bit_length() - 1) & 1
    bd = (iota >> int(d).bit_length() - 1) & 1
    return 1 - (bk ^ bd)


def _f2i(x):
    return lax.bitcast_convert_type(x, jnp.int32)


def _i2f(x):
    return lax.bitcast_convert_type(x, jnp.float32)


def _bisect_stage(u, d, swap):
    sh = u.shape
    n_u, lanes = sh[-2], sh[-1]
    y = u.reshape(sh[:-2] + (n_u // (2 * d), 2, d, lanes))
    a = y[..., 0:1, :, :]
    b = y[..., 1:2, :, :]
    mn = jnp.minimum(a, b)
    mx = jnp.maximum(a, b)
    lo, hi = (mx, mn) if swap else (mn, mx)
    return jnp.concatenate([lo, hi], axis=-3).reshape(sh)


def _bitonic_sort_values(x):
    n, lanes = x.shape
    k = 2
    while k < n:
        d = k // 2
        while d >= 1:
            v = x.reshape(n // (2 * k), 2, k, lanes)
            xa = _bisect_stage(v[:, 0], d, False)
            xd = _bisect_stage(v[:, 1], d, True)
            x = jnp.concatenate(
                [xa[:, None], xd[:, None]], axis=1
            ).reshape(n, lanes)
            d //= 2
        k *= 2
    d = n // 2
    while d >= 1:
        x = _bisect_stage(x, d, False)
        d //= 2
    return x


def _bitonic_argsort(keys):
    n = keys.shape[0]
    iota = lax.broadcasted_iota(jnp.int32, (n, 1), 0)
    vals = lax.broadcasted_iota(jnp.int32, keys.shape, 0)
    k = 2
    while k <= n:
        d = k // 2
        while d >= 1:
            m = _stage_mask(iota, k, d)
            pk = _partner(keys, d)
            pv = _partner(vals, d)
            ltk = (pk < keys).astype(jnp.int32)
            eqk = (pk == keys).astype(jnp.int32)
            ltv = (pv < vals).astype(jnp.int32)
            lt = ltk + eqk * ltv
            gt = 1 - lt
            sel = gt + m * (lt - gt)
            bk = _f2i(keys)
            keys = _i2f(bk + sel * (_f2i(pk) - bk))
            vals = vals + sel * (pv - vals)
            d //= 2
        k *= 2
    return keys, vals


def _prep_body(rs_ref, rt_ref, rindt_ref):
    rs = rs_ref[...]
    rt_ref[...] = rs.T
    _, rind = _bitonic_argsort(rs)
    m = rs.shape[0]
    lw = rs.shape[1] // 32
    io_l = lax.broadcasted_iota(jnp.int32, (rs.shape[1], 1), 0)
    rindt_ref[...] = rind.T + (io_l % lw) * m


def _prep(rslices, interpret=False):
    M, L = rslices.shape
    return pl.pallas_call(
        _prep_body,
        out_shape=[
            jax.ShapeDtypeStruct((L, M), jnp.float32),
            jax.ShapeDtypeStruct((L, M), jnp.int32),
        ],
        interpret=interpret,
    )(rslices)


def _proj_sort_body(x_ref, w_ref, out_ref):
    x = x_ref[0]
    xs = lax.dot_general(
        x, w_ref[...], (((1,), (1,)), ((), ())),
        preferred_element_type=jnp.float32,
    )
    xs = _bitonic_sort_values(xs)
    out_ref[0] = xs.T


def _proj_sort(X, w, b0=0, nb=None, interpret=False):
    B, N, D = X.shape
    L = w.shape[0]
    if nb is None:
        nb = B
    return pl.pallas_call(
        _proj_sort_body,
        grid=(nb,),
        in_specs=[
            pl.BlockSpec((1, N, D), lambda b: (b0 + b, 0, 0)),
            pl.BlockSpec((L, D), lambda b: (0, 0)),
        ],
        out_specs=pl.BlockSpec((1, L, N), lambda b: (b, 0, 0)),
        out_shape=jax.ShapeDtypeStruct((nb, L, N), jnp.float32),
        interpret=interpret,
    )(X, w)


def _sc_gather_sub(xsT, rt, rindt):
    B, L, N = xsT.shape
    info = plsc.get_sparse_core_info()
    NC, NS = info.num_cores, info.num_subcores
    NW = NC * NS
    LW = L // NW
    mesh = plsc.VectorSubcoreMesh(core_axis_name="c", subcore_axis_name="s")

    W16 = LW * N // 16
    UNROLL = 8

    @functools.partial(
        pl.kernel,
        out_type=jax.ShapeDtypeStruct((B, L * N), jnp.float32),
        mesh=mesh,
        compiler_params=pltpu.CompilerParams(needs_layout_passes=False),
        scratch_types=[
            pltpu.VMEM((LW * N,), jnp.float32),
            pltpu.VMEM((LW * N,), jnp.float32),
            pltpu.VMEM((LW * N,), jnp.float32),
            pltpu.VMEM((LW * N,), jnp.float32),
            pltpu.VMEM((LW * N,), jnp.float32),
            pltpu.VMEM((LW * N,), jnp.int32),
            pltpu.SemaphoreType.DMA,
            pltpu.SemaphoreType.DMA,
            pltpu.SemaphoreType.DMA,
            pltpu.SemaphoreType.DMA,
        ],
    )
    def k(xsT_hbm, rt_hbm, rindt_hbm, out_hbm,
          x0, x1, o0, o1, r_v, idx_v, si0, si1, so0, so1):
        wid = lax.axis_index("s") * NC + lax.axis_index("c")
        l0 = wid * LW
        sl = pl.ds(l0 * N, LW * N)
        pltpu.sync_copy(rt_hbm.at[sl], r_v)
        pltpu.sync_copy(rindt_hbm.at[sl], idx_v)

        def compute(x_v, o_v):
            def per_j(j, _):
                for u in range(UNROLL):
                    off = (j * UNROLL + u) * 16
                    g = plsc.load_gather(x_v, [idx_v[pl.ds(off, 16)]])
                    o_v[pl.ds(off, 16)] = r_v[pl.ds(off, 16)] - g
                return 0

            lax.fori_loop(0, W16 // UNROLL, per_j, 0)

        bufs = ((x0, o0, si0, so0), (x1, o1, si1, so1))
        pltpu.async_copy(xsT_hbm.at[0, sl], x0, si0)

        def step(g, _):
            for ph in range(2):
                x_v, o_v, si, so = bufs[ph]
                xn, _, sin, _ = bufs[1 - ph]
                b = g * 2 + ph
                pltpu.make_async_copy(xsT_hbm.at[b, sl], x_v, si).wait()

                @pl.when(b + 1 < B)
                def _():
                    pltpu.async_copy(xsT_hbm.at[b + 1, sl], xn, sin)

                @pl.when(g > 0)
                def _():
                    pltpu.make_async_copy(o_v, out_hbm.at[b, sl], so).wait()

                compute(x_v, o_v)
                pltpu.async_copy(o_v, out_hbm.at[b, sl], so)
            return 0

        lax.fori_loop(0, B // 2, step, 0)
        pltpu.make_async_copy(o0, out_hbm.at[0, sl], so0).wait()
        pltpu.make_async_copy(o1, out_hbm.at[0, sl], so1).wait()

    return k(
        xsT.reshape(B, L * N), rt.reshape(L * N), rindt.reshape(L * N)
    )


def kernel(X, ref_points, theta_v):
    W = theta_v / (jnp.linalg.norm(theta_v, axis=1, keepdims=True))
    Rslices = jnp.einsum('md,ld->ml', ref_points, W)
    rt, rindt = _prep(Rslices)
    B = X.shape[0]
    nch = 4
    bc = B // nch
    outs = []
    for c in range(nch):
        xsT_c = _proj_sort(X, W, b0=c * bc, nb=bc)
        outs.append(_sc_gather_sub(xsT_c, rt, rindt))
    return jnp.concatenate(outs, axis=0)


# speedup vs baseline: 1.0944x; 1.0944x over previous
import functools

import jax
import jax.numpy as jnp
from jax import lax
from jax.experimental import pallas as pl
from jax.experimental.pallas import tpu as pltpu
from jax.experimental.pallas import tpu_sc as plsc


def _partner(x, d):
    n = x.shape[0]
    tail = x.shape[1:]
    y = x.reshape((n // (2 * d), 2, d) + tail)
    y = jnp.concatenate([y[:, 1:2], y[:, 0:1]], axis=1)
    return y.reshape((n,) + tail)


def _stage_mask(iota, k, d):
    bk = (iota >> int(k).bit_length() - 1) & 1
    bd = (iota >> int(d).bit_length() - 1) & 1
    return 1 - (bk ^ bd)


def _f2i(x):
    return lax.bitcast_convert_type(x, jnp.int32)


def _i2f(x):
    return lax.bitcast_convert_type(x, jnp.float32)


def _bisect_stage(u, d, swap):
    sh = u.shape
    n_u, lanes = sh[-2], sh[-1]
    y = u.reshape(sh[:-2] + (n_u // (2 * d), 2, d, lanes))
    a = y[..., 0:1, :, :]
    b = y[..., 1:2, :, :]
    mn = jnp.minimum(a, b)
    mx = jnp.maximum(a, b)
    lo, hi = (mx, mn) if swap else (mn, mx)
    return jnp.concatenate([lo, hi], axis=-3).reshape(sh)


def _bisect_stage2(u, d, swap):
    sh = u.shape
    n_u, lanes = sh[-2], sh[-1]
    q = d // 2
    y = u.reshape(sh[:-2] + (n_u // (2 * d), 2, 2, q, lanes))
    a = y[..., 0, 0, :, :]
    b = y[..., 0, 1, :, :]
    c = y[..., 1, 0, :, :]
    e = y[..., 1, 1, :, :]
    mnf, mxf = (jnp.maximum, jnp.minimum) if swap else (jnp.minimum, jnp.maximum)
    mn1, mx1 = mnf(a, c), mxf(a, c)
    mn2, mx2 = mnf(b, e), mxf(b, e)
    o = [mnf(mn1, mn2), mxf(mn1, mn2), mnf(mx1, mx2), mxf(mx1, mx2)]
    return jnp.concatenate(
        [t[..., None, :, :] for t in o], axis=-3
    ).reshape(sh)


def _uniform_stages(u, dists, swap):
    i = 0
    while i < len(dists):
        if i + 1 < len(dists) and dists[i] >= 2:
            u = _bisect_stage2(u, dists[i], swap)
            i += 2
        else:
            u = _bisect_stage(u, dists[i], swap)
            i += 1
    return u


def _bitonic_sort_values(x):
    n, lanes = x.shape
    k = 2
    while k < n:
        dists = []
        d = k // 2
        while d >= 1:
            dists.append(d)
            d //= 2
        v = x.reshape(n // (2 * k), 2, k, lanes)
        xa = _uniform_stages(v[:, 0], dists, False)
        xd = _uniform_stages(v[:, 1], dists, True)
        x = jnp.concatenate([xa[:, None], xd[:, None]], axis=1).reshape(
            n, lanes
        )
        k *= 2
    dists = []
    d = n // 2
    while d >= 1:
        dists.append(d)
        d //= 2
    return _uniform_stages(x, dists, False)


def _bitonic_argsort(keys):
    n = keys.shape[0]
    iota = lax.broadcasted_iota(jnp.int32, (n, 1), 0)
    vals = lax.broadcasted_iota(jnp.int32, keys.shape, 0)
    k = 2
    while k <= n:
        d = k // 2
        while d >= 1:
            m = _stage_mask(iota, k, d)
            pk = _partner(keys, d)
            pv = _partner(vals, d)
            ltk = (pk < keys).astype(jnp.int32)
            eqk = (pk == keys).astype(jnp.int32)
            ltv = (pv < vals).astype(jnp.int32)
            lt = ltk + eqk * ltv
            gt = 1 - lt
            sel = gt + m * (lt - gt)
            bk = _f2i(keys)
            keys = _i2f(bk + sel * (_f2i(pk) - bk))
            vals = vals + sel * (pv - vals)
            d //= 2
        k *= 2
    return keys, vals


def _prep_body(rs_ref, rt_ref, rindt_ref):
    rs = rs_ref[...]
    rt_ref[...] = rs.T
    _, rind = _bitonic_argsort(rs)
    m = rs.shape[0]
    lw = rs.shape[1] // 32
    io_l = lax.broadcasted_iota(jnp.int32, (rs.shape[1], 1), 0)
    rindt_ref[...] = rind.T + (io_l % lw) * m


def _prep(rslices, interpret=False):
    M, L = rslices.shape
    return pl.pallas_call(
        _prep_body,
        out_shape=[
            jax.ShapeDtypeStruct((L, M), jnp.float32),
            jax.ShapeDtypeStruct((L, M), jnp.int32),
        ],
        interpret=interpret,
    )(rslices)


def _proj_sort_body(x_ref, w_ref, out_ref):
    x = x_ref[0]
    xs = lax.dot_general(
        x, w_ref[...], (((1,), (1,)), ((), ())),
        preferred_element_type=jnp.float32,
    )
    xs = _bitonic_sort_values(xs)
    out_ref[0] = xs.T


def _proj_sort(X, w, b0=0, nb=None, interpret=False):
    B, N, D = X.shape
    L = w.shape[0]
    if nb is None:
        nb = B
    return pl.pallas_call(
        _proj_sort_body,
        grid=(nb,),
        in_specs=[
            pl.BlockSpec((1, N, D), lambda b: (b0 + b, 0, 0)),
            pl.BlockSpec((L, D), lambda b: (0, 0)),
        ],
        out_specs=pl.BlockSpec((1, L, N), lambda b: (b, 0, 0)),
        out_shape=jax.ShapeDtypeStruct((nb, L, N), jnp.float32),
        interpret=interpret,
    )(X, w)


def _sc_gather_sub(xsT, rt, rindt):
    B, L, N = xsT.shape
    info = plsc.get_sparse_core_info()
    NC, NS = info.num_cores, info.num_subcores
    NW = NC * NS
    LW = L // NW
    mesh = plsc.VectorSubcoreMesh(core_axis_name="c", subcore_axis_name="s")

    W16 = LW * N // 16
    UNROLL = 8

    @functools.partial(
        pl.kernel,
        out_type=jax.ShapeDtypeStruct((B, L * N), jnp.float32),
        mesh=mesh,
        compiler_params=pltpu.CompilerParams(needs_layout_passes=False),
        scratch_types=[
            pltpu.VMEM((LW * N,), jnp.float32),
            pltpu.VMEM((LW * N,), jnp.float32),
            pltpu.VMEM((LW * N,), jnp.float32),
            pltpu.VMEM((LW * N,), jnp.float32),
            pltpu.VMEM((LW * N,), jnp.float32),
            pltpu.VMEM((LW * N,), jnp.int32),
            pltpu.SemaphoreType.DMA,
            pltpu.SemaphoreType.DMA,
            pltpu.SemaphoreType.DMA,
            pltpu.SemaphoreType.DMA,
        ],
    )
    def k(xsT_hbm, rt_hbm, rindt_hbm, out_hbm,
          x0, x1, o0, o1, r_v, idx_v, si0, si1, so0, so1):
        wid = lax.axis_index("s") * NC + lax.axis_index("c")
        l0 = wid * LW
        sl = pl.ds(l0 * N, LW * N)
        pltpu.sync_copy(rt_hbm.at[sl], r_v)
        pltpu.sync_copy(rindt_hbm.at[sl], idx_v)

        def compute(x_v, o_v):
            def per_j(j, _):
                for u in range(UNROLL):
                    off = (j * UNROLL + u) * 16
                    g = plsc.load_gather(x_v, [idx_v[pl.ds(off, 16)]])
                    o_v[pl.ds(off, 16)] = r_v[pl.ds(off, 16)] - g
                return 0

            lax.fori_loop(0, W16 // UNROLL, per_j, 0)

        bufs = ((x0, o0, si0, so0), (x1, o1, si1, so1))
        pltpu.async_copy(xsT_hbm.at[0, sl], x0, si0)

        def step(g, _):
            for ph in range(2):
                x_v, o_v, si, so = bufs[ph]
                xn, _, sin, _ = bufs[1 - ph]
                b = g * 2 + ph
                pltpu.make_async_copy(xsT_hbm.at[b, sl], x_v, si).wait()

                @pl.when(b + 1 < B)
                def _():
                    pltpu.async_copy(xsT_hbm.at[b + 1, sl], xn, sin)

                @pl.when(g > 0)
                def _():
                    pltpu.make_async_copy(o_v, out_hbm.at[b, sl], so).wait()

                compute(x_v, o_v)
                pltpu.async_copy(o_v, out_hbm.at[b, sl], so)
            return 0

        lax.fori_loop(0, B // 2, step, 0)
        pltpu.make_async_copy(o0, out_hbm.at[0, sl], so0).wait()
        pltpu.make_async_copy(o1, out_hbm.at[0, sl], so1).wait()

    return k(
        xsT.reshape(B, L * N), rt.reshape(L * N), rindt.reshape(L * N)
    )


def kernel(X, ref_points, theta_v):
    W = theta_v / (jnp.linalg.norm(theta_v, axis=1, keepdims=True))
    Rslices = jnp.einsum('md,ld->ml', ref_points, W)
    rt, rindt = _prep(Rslices)
    xsT = _proj_sort(X, W)
    return _sc_gather_sub(xsT, rt, rindt)


# speedup vs baseline: 1.1408x; 1.0424x over previous
import functools

import jax
import jax.numpy as jnp
from jax import lax
from jax.experimental import pallas as pl
from jax.experimental.pallas import tpu as pltpu
from jax.experimental.pallas import tpu_sc as plsc


def _partner(x, d):
    n = x.shape[0]
    tail = x.shape[1:]
    y = x.reshape((n // (2 * d), 2, d) + tail)
    y = jnp.concatenate([y[:, 1:2], y[:, 0:1]], axis=1)
    return y.reshape((n,) + tail)


def _stage_mask(iota, k, d):
    bk = (iota >> int(k).bit_length() - 1) & 1
    bd = (iota >> int(d).bit_length() - 1) & 1
    return 1 - (bk ^ bd)


def _f2i(x):
    return lax.bitcast_convert_type(x, jnp.int32)


def _i2f(x):
    return lax.bitcast_convert_type(x, jnp.float32)


def _bisect_stage(u, d, swap):
    sh = u.shape
    n_u, lanes = sh[-2], sh[-1]
    y = u.reshape(sh[:-2] + (n_u // (2 * d), 2, d, lanes))
    a = y[..., 0:1, :, :]
    b = y[..., 1:2, :, :]
    mn = jnp.minimum(a, b)
    mx = jnp.maximum(a, b)
    lo, hi = (mx, mn) if swap else (mn, mx)
    return jnp.concatenate([lo, hi], axis=-3).reshape(sh)


def _fused_stages(u, d, t, swap):
    sh = u.shape
    n_u, lanes = sh[-2], sh[-1]
    nslots = 1 << t
    q = d >> (t - 1)
    y = u.reshape(
        sh[:-2] + (n_u // (2 * d),) + (2,) * t + (q, lanes)
    )
    parts = []
    for idx in range(nslots):
        ix = tuple((idx >> (t - 1 - bb)) & 1 for bb in range(t))
        parts.append(y[(Ellipsis,) + ix + (slice(None), slice(None))])
    mnf, mxf = (
        (jnp.maximum, jnp.minimum) if swap else (jnp.minimum, jnp.maximum)
    )
    for s_ in range(t):
        step = 1 << (t - 1 - s_)
        newp = list(parts)
        for a_ in range(nslots):
            if (a_ // step) % 2 == 0:
                b_ = a_ + step
                newp[a_] = mnf(parts[a_], parts[b_])
                newp[b_] = mxf(parts[a_], parts[b_])
        parts = newp
    return jnp.concatenate(
        [p[..., None, :, :] for p in parts], axis=-3
    ).reshape(sh)


def _uniform_stages(u, dists, swap):
    i = 0
    while i < len(dists):
        left = len(dists) - i
        t = 1
        for cand in (3, 2):
            if left >= cand and dists[i] >= (1 << (cand - 1)):
                t = cand
                break
        if t == 1:
            u = _bisect_stage(u, dists[i], swap)
        else:
            u = _fused_stages(u, dists[i], t, swap)
        i += t
    return u


def _bitonic_sort_values(x):
    n, lanes = x.shape
    k = 2
    while k < n:
        dists = []
        d = k // 2
        while d >= 1:
            dists.append(d)
            d //= 2
        v = x.reshape(n // (2 * k), 2, k, lanes)
        xa = _uniform_stages(v[:, 0], dists, False)
        xd = _uniform_stages(v[:, 1], dists, True)
        x = jnp.concatenate([xa[:, None], xd[:, None]], axis=1).reshape(
            n, lanes
        )
        k *= 2
    dists = []
    d = n // 2
    while d >= 1:
        dists.append(d)
        d //= 2
    return _uniform_stages(x, dists, False)


def _bitonic_argsort(keys):
    n = keys.shape[0]
    iota = lax.broadcasted_iota(jnp.int32, (n, 1), 0)
    vals = lax.broadcasted_iota(jnp.int32, keys.shape, 0)
    k = 2
    while k <= n:
        d = k // 2
        while d >= 1:
            m = _stage_mask(iota, k, d)
            pk = _partner(keys, d)
            pv = _partner(vals, d)
            ltk = (pk < keys).astype(jnp.int32)
            eqk = (pk == keys).astype(jnp.int32)
            ltv = (pv < vals).astype(jnp.int32)
            lt = ltk + eqk * ltv
            gt = 1 - lt
            sel = gt + m * (lt - gt)
            bk = _f2i(keys)
            keys = _i2f(bk + sel * (_f2i(pk) - bk))
            vals = vals + sel * (pv - vals)
            d //= 2
        k *= 2
    return keys, vals


def _prep_body(rs_ref, rt_ref, rindt_ref):
    rs = rs_ref[...]
    rt_ref[...] = rs.T
    _, rind = _bitonic_argsort(rs)
    m = rs.shape[0]
    lw = rs.shape[1] // 32
    io_l = lax.broadcasted_iota(jnp.int32, (rs.shape[1], 1), 0)
    rindt_ref[...] = rind.T + (io_l % lw) * m


def _prep(rslices, interpret=False):
    M, L = rslices.shape
    return pl.pallas_call(
        _prep_body,
        out_shape=[
            jax.ShapeDtypeStruct((L, M), jnp.float32),
            jax.ShapeDtypeStruct((L, M), jnp.int32),
        ],
        interpret=interpret,
    )(rslices)


def _proj_sort_body(x_ref, w_ref, out_ref):
    x = x_ref[0]
    xs = lax.dot_general(
        x, w_ref[...], (((1,), (1,)), ((), ())),
        preferred_element_type=jnp.float32,
    )
    xs = _bitonic_sort_values(xs)
    out_ref[0] = xs.T


def _proj_sort(X, w, b0=0, nb=None, interpret=False):
    B, N, D = X.shape
    L = w.shape[0]
    if nb is None:
        nb = B
    return pl.pallas_call(
        _proj_sort_body,
        grid=(nb,),
        in_specs=[
            pl.BlockSpec((1, N, D), lambda b: (b0 + b, 0, 0)),
            pl.BlockSpec((L, D), lambda b: (0, 0)),
        ],
        out_specs=pl.BlockSpec((1, L, N), lambda b: (b, 0, 0)),
        out_shape=jax.ShapeDtypeStruct((nb, L, N), jnp.float32),
        interpret=interpret,
    )(X, w)


def _sc_gather_sub(xsT, rt, rindt):
    B, L, N = xsT.shape
    info = plsc.get_sparse_core_info()
    NC, NS = info.num_cores, info.num_subcores
    NW = NC * NS
    LW = L // NW
    mesh = plsc.VectorSubcoreMesh(core_axis_name="c", subcore_axis_name="s")

    W16 = LW * N // 16
    UNROLL = 16

    @functools.partial(
        pl.kernel,
        out_type=jax.ShapeDtypeStruct((B, L * N), jnp.float32),
        mesh=mesh,
        compiler_params=pltpu.CompilerParams(needs_layout_passes=False),
        scratch_types=[
            pltpu.VMEM((LW * N,), jnp.float32),
            pltpu.VMEM((LW * N,), jnp.float32),
            pltpu.VMEM((LW * N,), jnp.float32),
            pltpu.VMEM((LW * N,), jnp.float32),
            pltpu.VMEM((LW * N,), jnp.float32),
            pltpu.VMEM((LW * N,), jnp.int32),
            pltpu.SemaphoreType.DMA,
            pltpu.SemaphoreType.DMA,
            pltpu.SemaphoreType.DMA,
            pltpu.SemaphoreType.DMA,
        ],
    )
    def k(xsT_hbm, rt_hbm, rindt_hbm, out_hbm,
          x0, x1, o0, o1, r_v, idx_v, si0, si1, so0, so1):
        wid = lax.axis_index("s") * NC + lax.axis_index("c")
        l0 = wid * LW
        sl = pl.ds(l0 * N, LW * N)
        pltpu.sync_copy(rt_hbm.at[sl], r_v)
        pltpu.sync_copy(rindt_hbm.at[sl], idx_v)

        def compute(x_v, o_v):
            def per_j(j, _):
                for u in range(UNROLL):
                    off = (j * UNROLL + u) * 16
                    g = plsc.load_gather(x_v, [idx_v[pl.ds(off, 16)]])
                    o_v[pl.ds(off, 16)] = r_v[pl.ds(off, 16)] - g
                return 0

            lax.fori_loop(0, W16 // UNROLL, per_j, 0)

        bufs = ((x0, o0, si0, so0), (x1, o1, si1, so1))
        pltpu.async_copy(xsT_hbm.at[0, sl], x0, si0)

        def step(g, _):
            for ph in range(2):
                x_v, o_v, si, so = bufs[ph]
                xn, _, sin, _ = bufs[1 - ph]
                b = g * 2 + ph
                pltpu.make_async_copy(xsT_hbm.at[b, sl], x_v, si).wait()

                @pl.when(b + 1 < B)
                def _():
                    pltpu.async_copy(xsT_hbm.at[b + 1, sl], xn, sin)

                @pl.when(g > 0)
                def _():
                    pltpu.make_async_copy(o_v, out_hbm.at[b, sl], so).wait()

                compute(x_v, o_v)
                pltpu.async_copy(o_v, out_hbm.at[b, sl], so)
            return 0

        lax.fori_loop(0, B // 2, step, 0)
        pltpu.make_async_copy(o0, out_hbm.at[0, sl], so0).wait()
        pltpu.make_async_copy(o1, out_hbm.at[0, sl], so1).wait()

    return k(
        xsT.reshape(B, L * N), rt.reshape(L * N), rindt.reshape(L * N)
    )


def kernel(X, ref_points, theta_v):
    W = theta_v / (jnp.linalg.norm(theta_v, axis=1, keepdims=True))
    Rslices = jnp.einsum('md,ld->ml', ref_points, W)
    rt, rindt = _prep(Rslices)
    xsT = _proj_sort(X, W)
    return _sc_gather_sub(xsT, rt, rindt)


# speedup vs baseline: 1.2034x; 1.0548x over previous
import functools

import jax
import jax.numpy as jnp
from jax import lax
from jax.experimental import pallas as pl
from jax.experimental.pallas import tpu as pltpu
from jax.experimental.pallas import tpu_sc as plsc


def _partner(x, d):
    n = x.shape[0]
    tail = x.shape[1:]
    y = x.reshape((n // (2 * d), 2, d) + tail)
    y = jnp.concatenate([y[:, 1:2], y[:, 0:1]], axis=1)
    return y.reshape((n,) + tail)


def _stage_mask(iota, k, d):
    bk = (iota >> int(k).bit_length() - 1) & 1
    bd = (iota >> int(d).bit_length() - 1) & 1
    return 1 - (bk ^ bd)


def _f2i(x):
    return lax.bitcast_convert_type(x, jnp.int32)


def _i2f(x):
    return lax.bitcast_convert_type(x, jnp.float32)


def _bisect_stage(u, d, swap):
    sh = u.shape
    n_u, lanes = sh[-2], sh[-1]
    y = u.reshape(sh[:-2] + (n_u // (2 * d), 2, d, lanes))
    a = y[..., 0:1, :, :]
    b = y[..., 1:2, :, :]
    mn = jnp.minimum(a, b)
    mx = jnp.maximum(a, b)
    lo, hi = (mx, mn) if swap else (mn, mx)
    return jnp.concatenate([lo, hi], axis=-3).reshape(sh)


def _fused_stages(u, d, t, swap):
    sh = u.shape
    n_u, lanes = sh[-2], sh[-1]
    nslots = 1 << t
    q = d >> (t - 1)
    y = u.reshape(
        sh[:-2] + (n_u // (2 * d),) + (2,) * t + (q, lanes)
    )
    parts = []
    for idx in range(nslots):
        ix = tuple((idx >> (t - 1 - bb)) & 1 for bb in range(t))
        parts.append(y[(Ellipsis,) + ix + (slice(None), slice(None))])
    mnf, mxf = (
        (jnp.maximum, jnp.minimum) if swap else (jnp.minimum, jnp.maximum)
    )
    for s_ in range(t):
        step = 1 << (t - 1 - s_)
        newp = list(parts)
        for a_ in range(nslots):
            if (a_ // step) % 2 == 0:
                b_ = a_ + step
                newp[a_] = mnf(parts[a_], parts[b_])
                newp[b_] = mxf(parts[a_], parts[b_])
        parts = newp
    return jnp.concatenate(
        [p[..., None, :, :] for p in parts], axis=-3
    ).reshape(sh)


def _uniform_stages(u, dists, swap):
    i = 0
    while i < len(dists):
        left = len(dists) - i
        t = 1
        for cand in (4, 3, 2):
            if left >= cand and dists[i] >= (1 << (cand - 1)):
                t = cand
                break
        if t == 1:
            u = _bisect_stage(u, dists[i], swap)
        else:
            u = _fused_stages(u, dists[i], t, swap)
        i += t
    return u


def _bitonic_sort_values(x):
    n, lanes = x.shape
    k = 2
    while k < n:
        dists = []
        d = k // 2
        while d >= 1:
            dists.append(d)
            d //= 2
        v = x.reshape(n // (2 * k), 2, k, lanes)
        xa = _uniform_stages(v[:, 0], dists, False)
        xd = _uniform_stages(v[:, 1], dists, True)
        x = jnp.concatenate([xa[:, None], xd[:, None]], axis=1).reshape(
            n, lanes
        )
        k *= 2
    dists = []
    d = n // 2
    while d >= 1:
        dists.append(d)
        d //= 2
    return _uniform_stages(x, dists, False)


def _bitonic_argsort(keys):
    n = keys.shape[0]
    iota = lax.broadcasted_iota(jnp.int32, (n, 1), 0)
    vals = lax.broadcasted_iota(jnp.int32, keys.shape, 0)
    k = 2
    while k <= n:
        d = k // 2
        while d >= 1:
            m = _stage_mask(iota, k, d)
            pk = _partner(keys, d)
            pv = _partner(vals, d)
            ltk = (pk < keys).astype(jnp.int32)
            eqk = (pk == keys).astype(jnp.int32)
            ltv = (pv < vals).astype(jnp.int32)
            lt = ltk + eqk * ltv
            gt = 1 - lt
            sel = gt + m * (lt - gt)
            bk = _f2i(keys)
            keys = _i2f(bk + sel * (_f2i(pk) - bk))
            vals = vals + sel * (pv - vals)
            d //= 2
        k *= 2
    return keys, vals


def _prep_body(rs_ref, rt_ref, rindt_ref):
    rs = rs_ref[...]
    rt_ref[...] = rs.T
    _, rind = _bitonic_argsort(rs)
    m = rs.shape[0]
    lw = rs.shape[1] // 32
    io_l = lax.broadcasted_iota(jnp.int32, (rs.shape[1], 1), 0)
    rindt_ref[...] = rind.T + (io_l % lw) * m


def _prep(rslices, interpret=False):
    M, L = rslices.shape
    return pl.pallas_call(
        _prep_body,
        out_shape=[
            jax.ShapeDtypeStruct((L, M), jnp.float32),
            jax.ShapeDtypeStruct((L, M), jnp.int32),
        ],
        interpret=interpret,
    )(rslices)


def _proj_sort_body(x_ref, w_ref, out_ref):
    x = x_ref[0]
    xs = lax.dot_general(
        x, w_ref[...], (((1,), (1,)), ((), ())),
        preferred_element_type=jnp.float32,
    )
    xs = _bitonic_sort_values(xs)
    out_ref[0] = xs.T


def _proj_sort(X, w, b0=0, nb=None, interpret=False):
    B, N, D = X.shape
    L = w.shape[0]
    if nb is None:
        nb = B
    return pl.pallas_call(
        _proj_sort_body,
        grid=(nb,),
        in_specs=[
            pl.BlockSpec((1, N, D), lambda b: (b0 + b, 0, 0)),
            pl.BlockSpec((L, D), lambda b: (0, 0)),
        ],
        out_specs=pl.BlockSpec((1, L, N), lambda b: (b, 0, 0)),
        out_shape=jax.ShapeDtypeStruct((nb, L, N), jnp.float32),
        interpret=interpret,
    )(X, w)


def _sc_gather_sub(xsT, rt, rindt):
    B, L, N = xsT.shape
    info = plsc.get_sparse_core_info()
    NC, NS = info.num_cores, info.num_subcores
    NW = NC * NS
    LW = L // NW
    mesh = plsc.VectorSubcoreMesh(core_axis_name="c", subcore_axis_name="s")

    W16 = LW * N // 16
    UNROLL = 16

    @functools.partial(
        pl.kernel,
        out_type=jax.ShapeDtypeStruct((B, L * N), jnp.float32),
        mesh=mesh,
        compiler_params=pltpu.CompilerParams(needs_layout_passes=False),
        scratch_types=[
            pltpu.VMEM((LW * N,), jnp.float32),
            pltpu.VMEM((LW * N,), jnp.float32),
            pltpu.VMEM((LW * N,), jnp.float32),
            pltpu.VMEM((LW * N,), jnp.float32),
            pltpu.VMEM((LW * N,), jnp.float32),
            pltpu.VMEM((LW * N,), jnp.int32),
            pltpu.SemaphoreType.DMA,
            pltpu.SemaphoreType.DMA,
            pltpu.SemaphoreType.DMA,
            pltpu.SemaphoreType.DMA,
        ],
    )
    def k(xsT_hbm, rt_hbm, rindt_hbm, out_hbm,
          x0, x1, o0, o1, r_v, idx_v, si0, si1, so0, so1):
        wid = lax.axis_index("s") * NC + lax.axis_index("c")
        l0 = wid * LW
        sl = pl.ds(l0 * N, LW * N)
        pltpu.sync_copy(rt_hbm.at[sl], r_v)
        pltpu.sync_copy(rindt_hbm.at[sl], idx_v)

        def compute(x_v, o_v):
            def per_j(j, _):
                for u in range(UNROLL):
                    off = (j * UNROLL + u) * 16
                    g = plsc.load_gather(x_v, [idx_v[pl.ds(off, 16)]])
                    o_v[pl.ds(off, 16)] = r_v[pl.ds(off, 16)] - g
                return 0

            lax.fori_loop(0, W16 // UNROLL, per_j, 0)

        bufs = ((x0, o0, si0, so0), (x1, o1, si1, so1))
        pltpu.async_copy(xsT_hbm.at[0, sl], x0, si0)

        def step(g, _):
            for ph in range(2):
                x_v, o_v, si, so = bufs[ph]
                xn, _, sin, _ = bufs[1 - ph]
                b = g * 2 + ph
                pltpu.make_async_copy(xsT_hbm.at[b, sl], x_v, si).wait()

                @pl.when(b + 1 < B)
                def _():
                    pltpu.async_copy(xsT_hbm.at[b + 1, sl], xn, sin)

                @pl.when(g > 0)
                def _():
                    pltpu.make_async_copy(o_v, out_hbm.at[b, sl], so).wait()

                compute(x_v, o_v)
                pltpu.async_copy(o_v, out_hbm.at[b, sl], so)
            return 0

        lax.fori_loop(0, B // 2, step, 0)
        pltpu.make_async_copy(o0, out_hbm.at[0, sl], so0).wait()
        pltpu.make_async_copy(o1, out_hbm.at[0, sl], so1).wait()

    return k(
        xsT.reshape(B, L * N), rt.reshape(L * N), rindt.reshape(L * N)
    )


def kernel(X, ref_points, theta_v):
    W = theta_v / (jnp.linalg.norm(theta_v, axis=1, keepdims=True))
    Rslices = jnp.einsum('md,ld->ml', ref_points, W)
    rt, rindt = _prep(Rslices)
    xsT = _proj_sort(X, W)
    return _sc_gather_sub(xsT, rt, rindt)


# speedup vs baseline: 1.2169x; 1.0113x over previous
import functools

import jax
import jax.numpy as jnp
from jax import lax
from jax.experimental import pallas as pl
from jax.experimental.pallas import tpu as pltpu
from jax.experimental.pallas import tpu_sc as plsc


def _partner(x, d):
    n = x.shape[0]
    tail = x.shape[1:]
    y = x.reshape((n // (2 * d), 2, d) + tail)
    y = jnp.concatenate([y[:, 1:2], y[:, 0:1]], axis=1)
    return y.reshape((n,) + tail)


def _stage_mask(iota, k, d):
    bk = (iota >> int(k).bit_length() - 1) & 1
    bd = (iota >> int(d).bit_length() - 1) & 1
    return 1 - (bk ^ bd)


def _f2i(x):
    return lax.bitcast_convert_type(x, jnp.int32)


def _i2f(x):
    return lax.bitcast_convert_type(x, jnp.float32)


def _bisect_stage(u, d, swap):
    sh = u.shape
    n_u, lanes = sh[-2], sh[-1]
    y = u.reshape(sh[:-2] + (n_u // (2 * d), 2, d, lanes))
    a = y[..., 0:1, :, :]
    b = y[..., 1:2, :, :]
    mn = jnp.minimum(a, b)
    mx = jnp.maximum(a, b)
    lo, hi = (mx, mn) if swap else (mn, mx)
    return jnp.concatenate([lo, hi], axis=-3).reshape(sh)


def _fused_stages(u, d, t, swap):
    sh = u.shape
    n_u, lanes = sh[-2], sh[-1]
    nslots = 1 << t
    q = d >> (t - 1)
    y = u.reshape(
        sh[:-2] + (n_u // (2 * d),) + (2,) * t + (q, lanes)
    )
    parts = []
    for idx in range(nslots):
        ix = tuple((idx >> (t - 1 - bb)) & 1 for bb in range(t))
        parts.append(y[(Ellipsis,) + ix + (slice(None), slice(None))])
    mnf, mxf = (
        (jnp.maximum, jnp.minimum) if swap else (jnp.minimum, jnp.maximum)
    )
    for s_ in range(t):
        step = 1 << (t - 1 - s_)
        newp = list(parts)
        for a_ in range(nslots):
            if (a_ // step) % 2 == 0:
                b_ = a_ + step
                newp[a_] = mnf(parts[a_], parts[b_])
                newp[b_] = mxf(parts[a_], parts[b_])
        parts = newp
    return jnp.concatenate(
        [p[..., None, :, :] for p in parts], axis=-3
    ).reshape(sh)


def _uniform_stages(u, dists, swap):
    i = 0
    while i < len(dists):
        left = len(dists) - i
        t = 1
        for cand in (5, 4, 3, 2):
            if left >= cand and dists[i] >= (1 << (cand - 1)):
                t = cand
                break
        if t == 1:
            u = _bisect_stage(u, dists[i], swap)
        else:
            u = _fused_stages(u, dists[i], t, swap)
        i += t
    return u


def _bitonic_sort_values(x):
    n, lanes = x.shape
    k = 2
    while k < n:
        dists = []
        d = k // 2
        while d >= 1:
            dists.append(d)
            d //= 2
        v = x.reshape(n // (2 * k), 2, k, lanes)
        xa = _uniform_stages(v[:, 0], dists, False)
        xd = _uniform_stages(v[:, 1], dists, True)
        x = jnp.concatenate([xa[:, None], xd[:, None]], axis=1).reshape(
            n, lanes
        )
        k *= 2
    dists = []
    d = n // 2
    while d >= 1:
        dists.append(d)
        d //= 2
    return _uniform_stages(x, dists, False)


def _bitonic_argsort(keys):
    n = keys.shape[0]
    iota = lax.broadcasted_iota(jnp.int32, (n, 1), 0)
    vals = lax.broadcasted_iota(jnp.int32, keys.shape, 0)
    k = 2
    while k <= n:
        d = k // 2
        while d >= 1:
            m = _stage_mask(iota, k, d)
            pk = _partner(keys, d)
            pv = _partner(vals, d)
            ltk = (pk < keys).astype(jnp.int32)
            eqk = (pk == keys).astype(jnp.int32)
            ltv = (pv < vals).astype(jnp.int32)
            lt = ltk + eqk * ltv
            gt = 1 - lt
            sel = gt + m * (lt - gt)
            bk = _f2i(keys)
            keys = _i2f(bk + sel * (_f2i(pk) - bk))
            vals = vals + sel * (pv - vals)
            d //= 2
        k *= 2
    return keys, vals


def _prep_body(rs_ref, rt_ref, rindt_ref):
    rs = rs_ref[...]
    rt_ref[...] = rs.T
    _, rind = _bitonic_argsort(rs)
    m = rs.shape[0]
    lw = rs.shape[1] // 32
    io_l = lax.broadcasted_iota(jnp.int32, (rs.shape[1], 1), 0)
    rindt_ref[...] = rind.T + (io_l % lw) * m


def _prep(rslices, interpret=False):
    M, L = rslices.shape
    return pl.pallas_call(
        _prep_body,
        out_shape=[
            jax.ShapeDtypeStruct((L, M), jnp.float32),
            jax.ShapeDtypeStruct((L, M), jnp.int32),
        ],
        interpret=interpret,
    )(rslices)


def _proj_sort_body(x_ref, w_ref, out_ref):
    x = x_ref[0]
    xs = lax.dot_general(
        x, w_ref[...], (((1,), (1,)), ((), ())),
        preferred_element_type=jnp.float32,
    )
    xs = _bitonic_sort_values(xs)
    out_ref[0] = xs.T


def _proj_sort(X, w, b0=0, nb=None, interpret=False):
    B, N, D = X.shape
    L = w.shape[0]
    if nb is None:
        nb = B
    return pl.pallas_call(
        _proj_sort_body,
        grid=(nb,),
        in_specs=[
            pl.BlockSpec((1, N, D), lambda b: (b0 + b, 0, 0)),
            pl.BlockSpec((L, D), lambda b: (0, 0)),
        ],
        out_specs=pl.BlockSpec((1, L, N), lambda b: (b, 0, 0)),
        out_shape=jax.ShapeDtypeStruct((nb, L, N), jnp.float32),
        interpret=interpret,
    )(X, w)


def _sc_gather_sub(xsT, rt, rindt):
    B, L, N = xsT.shape
    info = plsc.get_sparse_core_info()
    NC, NS = info.num_cores, info.num_subcores
    NW = NC * NS
    LW = L // NW
    mesh = plsc.VectorSubcoreMesh(core_axis_name="c", subcore_axis_name="s")

    W16 = LW * N // 16
    UNROLL = 16

    @functools.partial(
        pl.kernel,
        out_type=jax.ShapeDtypeStruct((B, L * N), jnp.float32),
        mesh=mesh,
        compiler_params=pltpu.CompilerParams(needs_layout_passes=False),
        scratch_types=[
            pltpu.VMEM((LW * N,), jnp.float32),
            pltpu.VMEM((LW * N,), jnp.float32),
            pltpu.VMEM((LW * N,), jnp.float32),
            pltpu.VMEM((LW * N,), jnp.float32),
            pltpu.VMEM((LW * N,), jnp.float32),
            pltpu.VMEM((LW * N,), jnp.int32),
            pltpu.SemaphoreType.DMA,
            pltpu.SemaphoreType.DMA,
            pltpu.SemaphoreType.DMA,
            pltpu.SemaphoreType.DMA,
        ],
    )
    def k(xsT_hbm, rt_hbm, rindt_hbm, out_hbm,
          x0, x1, o0, o1, r_v, idx_v, si0, si1, so0, so1):
        wid = lax.axis_index("s") * NC + lax.axis_index("c")
        l0 = wid * LW
        sl = pl.ds(l0 * N, LW * N)
        pltpu.sync_copy(rt_hbm.at[sl], r_v)
        pltpu.sync_copy(rindt_hbm.at[sl], idx_v)

        def compute(x_v, o_v):
            def per_j(j, _):
                for u in range(UNROLL):
                    off = (j * UNROLL + u) * 16
                    g = plsc.load_gather(x_v, [idx_v[pl.ds(off, 16)]])
                    o_v[pl.ds(off, 16)] = r_v[pl.ds(off, 16)] - g
                return 0

            lax.fori_loop(0, W16 // UNROLL, per_j, 0)

        bufs = ((x0, o0, si0, so0), (x1, o1, si1, so1))
        pltpu.async_copy(xsT_hbm.at[0, sl], x0, si0)

        def step(g, _):
            for ph in range(2):
                x_v, o_v, si, so = bufs[ph]
                xn, _, sin, _ = bufs[1 - ph]
                b = g * 2 + ph
                pltpu.make_async_copy(xsT_hbm.at[b, sl], x_v, si).wait()

                @pl.when(b + 1 < B)
                def _():
                    pltpu.async_copy(xsT_hbm.at[b + 1, sl], xn, sin)

                @pl.when(g > 0)
                def _():
                    pltpu.make_async_copy(o_v, out_hbm.at[b, sl], so).wait()

                compute(x_v, o_v)
                pltpu.async_copy(o_v, out_hbm.at[b, sl], so)
            return 0

        lax.fori_loop(0, B // 2, step, 0)
        pltpu.make_async_copy(o0, out_hbm.at[0, sl], so0).wait()
        pltpu.make_async_copy(o1, out_hbm.at[0, sl], so1).wait()

    return k(
        xsT.reshape(B, L * N), rt.reshape(L * N), rindt.reshape(L * N)
    )


def kernel(X, ref_points, theta_v):
    W = theta_v / (jnp.linalg.norm(theta_v, axis=1, keepdims=True))
    Rslices = jnp.einsum('md,ld->ml', ref_points, W)
    rt, rindt = _prep(Rslices)
    xsT = _proj_sort(X, W)
    return _sc_gather_sub(xsT, rt, rindt)
